# Initial kernel scaffold; baseline (speedup 1.0000x reference)
import jax
import jax.numpy as jnp
from jax.experimental import pallas as pl


def kernel(q, r, k_table, v_table):
    raise NotImplementedError("write your pallas kernel here")


# speedup vs baseline: 11.9555x; 11.9555x over previous
import functools

import jax
import jax.numpy as jnp
from jax import lax
from jax.experimental import pallas as pl
from jax.experimental.pallas import tpu as pltpu
from jax.experimental.pallas import tpu_sc as plsc

NC = 2
NS = 16
L = 16
NW = NC * NS


def _dkvmn_body(num_q, b_per_w, chunk, n_chunks,
                q_hbm, r_hbm, kt_hbm, vt_hbm, ko_hbm, vo_hbm,
                q_v, qr_v, k_rows, v_rows, sem_k, sem_v):
    wid = lax.axis_index("s") * NC + lax.axis_index("c")
    base = wid * b_per_w
    n_slices = chunk // 128

    def body(g, carry):
        off = base + g * chunk
        pltpu.sync_copy(q_hbm.at[pl.ds(off, chunk)], q_v)
        pltpu.sync_copy(r_hbm.at[pl.ds(off, chunk)], qr_v)
        for i in range(chunk // L):
            s = pl.ds(i * L, L)
            qr_v[s] = q_v[s] + qr_v[s] * num_q
        handles = []
        for j in range(n_slices):
            s = pl.ds(j * 128, 128)
            handles.append(pltpu.async_copy(kt_hbm.at[q_v.at[s]],
                                            k_rows.at[s], sem_k))
            handles.append(pltpu.async_copy(vt_hbm.at[qr_v.at[s]],
                                            v_rows.at[s], sem_v))
        for h in handles:
            h.wait()
        pltpu.sync_copy(k_rows, ko_hbm.at[pl.ds(off, chunk)])
        pltpu.sync_copy(v_rows, vo_hbm.at[pl.ds(off, chunk)])
        return carry

    lax.fori_loop(0, n_chunks, body, 0)


@functools.partial(jax.jit, static_argnums=(4, 5, 6))
def _dkvmn_sc(q_flat, r_flat, k_table, v_table, num_q, dim, chunk):
    b = q_flat.shape[0]
    b_per_w = b // NW
    n_chunks = b_per_w // chunk
    mesh = plsc.VectorSubcoreMesh(core_axis_name="c", subcore_axis_name="s")
    body = functools.partial(_dkvmn_body, num_q, b_per_w, chunk, n_chunks)
    f = pl.kernel(
        body,
        out_type=(
            jax.ShapeDtypeStruct((b, dim), jnp.float32),
            jax.ShapeDtypeStruct((b, dim), jnp.float32),
        ),
        mesh=mesh,
        scratch_types=[
            pltpu.VMEM((chunk,), jnp.int32),
            pltpu.VMEM((chunk,), jnp.int32),
            pltpu.VMEM((chunk, dim), jnp.float32),
            pltpu.VMEM((chunk, dim), jnp.float32),
            pltpu.SemaphoreType.DMA,
            pltpu.SemaphoreType.DMA,
        ],
        compiler_params=pltpu.CompilerParams(use_tc_tiling_on_sc=False),
    )
    return f(q_flat, r_flat, k_table, v_table)


def kernel(q, r, k_table, v_table):
    batch, seq = q.shape
    num_q, dim = k_table.shape
    q_flat = q.reshape(-1).astype(jnp.int32)
    r_flat = r.reshape(-1).astype(jnp.int32)
    k_flat, v_flat = _dkvmn_sc(q_flat, r_flat, k_table, v_table,
                               num_q, dim, 512)
    return (k_flat.reshape(batch, seq, dim), v_flat.reshape(batch, seq, dim))


# speedup vs baseline: 12.6978x; 1.0621x over previous
import functools

import jax
import jax.numpy as jnp
from jax import lax
from jax.experimental import pallas as pl
from jax.experimental.pallas import tpu as pltpu
from jax.experimental.pallas import tpu_sc as plsc

NC = 2
NS = 16
L = 16
NW = NC * NS
SL = 128


def _dkvmn_body(num_q, b_per_w, chunk, n_chunks,
                q_hbm, r_hbm, kt_hbm, vt_hbm, ko_hbm, vo_hbm,
                q_v, r_v, k_rows, v_rows, sem_g, sem_out, sem_idx):
    wid = lax.axis_index("s") * NC + lax.axis_index("c")
    base = wid * b_per_w
    n_slices = chunk // SL

    def load_idx(g, p):
        off = base + g * chunk
        return [pltpu.async_copy(q_hbm.at[pl.ds(off, chunk)], q_v.at[p],
                                 sem_idx),
                pltpu.async_copy(r_hbm.at[pl.ds(off, chunk)], r_v.at[p],
                                 sem_idx)]

    def compute_qr(p):
        for i in range(chunk // L):
            s = pl.ds(i * L, L)
            r_v[p, s] = q_v[p, s] + r_v[p, s] * num_q

    def fire_gathers(p):
        hs = []
        for j in range(n_slices):
            s = pl.ds(j * SL, SL)
            hs.append(pltpu.async_copy(kt_hbm.at[q_v.at[p].at[s]],
                                       k_rows.at[p].at[s], sem_g))
            hs.append(pltpu.async_copy(vt_hbm.at[r_v.at[p].at[s]],
                                       v_rows.at[p].at[s], sem_g))
        return hs

    def fire_writes(g, p):
        off = base + g * chunk
        return [pltpu.async_copy(k_rows.at[p], ko_hbm.at[pl.ds(off, chunk)],
                                 sem_out),
                pltpu.async_copy(v_rows.at[p], vo_hbm.at[pl.ds(off, chunk)],
                                 sem_out)]

    def step(g, p, first, last):
        compute_qr(p)
        hs = fire_gathers(p)
        if not first:
            hs += fire_writes(g - 1, 1 - p)
        if not last:
            hs += load_idx(g + 1, 1 - p)
        for h in hs:
            h.wait()

    for h in load_idx(0, 0):
        h.wait()
    step(0, 0, True, False)

    def body(i, carry):
        g = 1 + 2 * i
        step(g, 1, False, False)
        step(g + 1, 0, False, False)
        return carry

    lax.fori_loop(0, (n_chunks - 2) // 2, body, 0)

    step(n_chunks - 1, 1, False, True)
    for h in fire_writes(n_chunks - 1, 1):
        h.wait()


@functools.partial(jax.jit, static_argnums=(4, 5, 6))
def _dkvmn_sc(q_flat, r_flat, k_table, v_table, num_q, dim, chunk):
    b = q_flat.shape[0]
    b_per_w = b // NW
    n_chunks = b_per_w // chunk
    mesh = plsc.VectorSubcoreMesh(core_axis_name="c", subcore_axis_name="s")
    body = functools.partial(_dkvmn_body, num_q, b_per_w, chunk, n_chunks)
    f = pl.kernel(
        body,
        out_type=(
            jax.ShapeDtypeStruct((b, dim), jnp.float32),
            jax.ShapeDtypeStruct((b, dim), jnp.float32),
        ),
        mesh=mesh,
        scratch_types=[
            pltpu.VMEM((2, chunk), jnp.int32),
            pltpu.VMEM((2, chunk), jnp.int32),
            pltpu.VMEM((2, chunk, dim), jnp.float32),
            pltpu.VMEM((2, chunk, dim), jnp.float32),
            pltpu.SemaphoreType.DMA,
            pltpu.SemaphoreType.DMA,
            pltpu.SemaphoreType.DMA,
        ],
        compiler_params=pltpu.CompilerParams(use_tc_tiling_on_sc=False),
    )
    return f(q_flat, r_flat, k_table, v_table)


def kernel(q, r, k_table, v_table):
    batch, seq = q.shape
    num_q, dim = k_table.shape
    q_flat = q.reshape(-1).astype(jnp.int32)
    r_flat = r.reshape(-1).astype(jnp.int32)
    k_flat, v_flat = _dkvmn_sc(q_flat, r_flat, k_table, v_table,
                               num_q, dim, 256)
    return (k_flat.reshape(batch, seq, dim), v_flat.reshape(batch, seq, dim))


# speedup vs baseline: 12.7147x; 1.0013x over previous
import functools

import jax
import jax.numpy as jnp
from jax import lax
from jax.experimental import pallas as pl
from jax.experimental.pallas import tpu as pltpu
from jax.experimental.pallas import tpu_sc as plsc

NC = 2
NS = 16
L = 16
NW = NC * NS
SL = 256


def _dkvmn_body(num_q, b_per_w, chunk, n_chunks,
                q_hbm, r_hbm, kt_hbm, vt_hbm, ko_hbm, vo_hbm,
                q_v, r_v, k_rows, v_rows, sem_g, sem_out, sem_idx):
    wid = lax.axis_index("s") * NC + lax.axis_index("c")
    base = wid * b_per_w
    n_slices = chunk // SL

    def load_idx(g, p):
        off = base + g * chunk
        return [pltpu.async_copy(q_hbm.at[pl.ds(off, chunk)], q_v.at[p],
                                 sem_idx),
                pltpu.async_copy(r_hbm.at[pl.ds(off, chunk)], r_v.at[p],
                                 sem_idx)]

    def compute_qr(p):
        for i in range(chunk // L):
            s = pl.ds(i * L, L)
            r_v[p, s] = q_v[p, s] + r_v[p, s] * num_q

    def fire_gathers(p):
        hs = []
        for j in range(n_slices):
            s = pl.ds(j * SL, SL)
            hs.append(pltpu.async_copy(kt_hbm.at[q_v.at[p].at[s]],
                                       k_rows.at[p].at[s], sem_g))
            hs.append(pltpu.async_copy(vt_hbm.at[r_v.at[p].at[s]],
                                       v_rows.at[p].at[s], sem_g))
        return hs

    def fire_writes(g, p):
        off = base + g * chunk
        return [pltpu.async_copy(k_rows.at[p], ko_hbm.at[pl.ds(off, chunk)],
                                 sem_out),
                pltpu.async_copy(v_rows.at[p], vo_hbm.at[pl.ds(off, chunk)],
                                 sem_out)]

    def step(g, p, first, last):
        compute_qr(p)
        hs = fire_gathers(p)
        if not first:
            hs += fire_writes(g - 1, 1 - p)
        if not last:
            hs += load_idx(g + 1, 1 - p)
        for h in hs:
            h.wait()

    for h in load_idx(0, 0):
        h.wait()
    step(0, 0, True, False)

    def body(i, carry):
        g = 1 + 2 * i
        step(g, 1, False, False)
        step(g + 1, 0, False, False)
        return carry

    lax.fori_loop(0, (n_chunks - 2) // 2, body, 0)

    step(n_chunks - 1, 1, False, True)
    for h in fire_writes(n_chunks - 1, 1):
        h.wait()


@functools.partial(jax.jit, static_argnums=(4, 5, 6))
def _dkvmn_sc(q_flat, r_flat, k_table, v_table, num_q, dim, chunk):
    b = q_flat.shape[0]
    b_per_w = b // NW
    n_chunks = b_per_w // chunk
    mesh = plsc.VectorSubcoreMesh(core_axis_name="c", subcore_axis_name="s")
    body = functools.partial(_dkvmn_body, num_q, b_per_w, chunk, n_chunks)
    f = pl.kernel(
        body,
        out_type=(
            jax.ShapeDtypeStruct((b, dim), jnp.float32),
            jax.ShapeDtypeStruct((b, dim), jnp.float32),
        ),
        mesh=mesh,
        scratch_types=[
            pltpu.VMEM((2, chunk), jnp.int32),
            pltpu.VMEM((2, chunk), jnp.int32),
            pltpu.VMEM((2, chunk, dim), jnp.float32),
            pltpu.VMEM((2, chunk, dim), jnp.float32),
            pltpu.SemaphoreType.DMA,
            pltpu.SemaphoreType.DMA,
            pltpu.SemaphoreType.DMA,
        ],
        compiler_params=pltpu.CompilerParams(use_tc_tiling_on_sc=False),
    )
    return f(q_flat, r_flat, k_table, v_table)


def kernel(q, r, k_table, v_table):
    batch, seq = q.shape
    num_q, dim = k_table.shape
    q_flat = q.reshape(-1).astype(jnp.int32)
    r_flat = r.reshape(-1).astype(jnp.int32)
    k_flat, v_flat = _dkvmn_sc(q_flat, r_flat, k_table, v_table,
                               num_q, dim, 256)
    return (k_flat.reshape(batch, seq, dim), v_flat.reshape(batch, seq, dim))
